# baseline (speedup 1.0000x reference)
import functools

import jax
import jax.numpy as jnp
from jax import lax
from jax.experimental import pallas as pl
from jax.experimental.pallas import tpu as pltpu
from jax.experimental.pallas import tpu_sc as plsc

N = 10000
E = 320000
NC = 2
NS = 16
LANES = 16
NW = NC * NS
CHUNK = 128
CPT = 80
EPT = CHUNK * CPT
EP = EPT * NW
RPT = 640
NACC = RPT * NS

def _zero_buf(buf, width):
    @pl.loop(0, CHUNK)
    def _(r):
        for c in range(width // LANES):
            buf[r, pl.ds(c * LANES, LANES)] = jnp.zeros((LANES,), jnp.float32)


def _fill_ones(buf):
    @pl.loop(0, CHUNK)
    def _(r):
        buf[r, :] = jnp.full((LANES,), 1.0, jnp.float32)


def _copy_out_stripe(acc_sh, buf, out_hbm, cid, sid, width):
    for c in range(RPT // CHUNK):
        row = sid * RPT + c * CHUNK
        pltpu.sync_copy(acc_sh.at[pl.ds(row, CHUNK)], buf)
        pltpu.sync_copy(buf, out_hbm.at[cid, pl.ds(row, CHUNK)])


@functools.cache
def _make_sc_degree():
    @functools.partial(
        pl.kernel,
        out_type=jax.ShapeDtypeStruct((NC, NACC, LANES), jnp.float32),
        mesh=plsc.VectorSubcoreMesh(core_axis_name="c", subcore_axis_name="s"),
        scratch_types=[
            pltpu.VMEM((CPT, CHUNK), jnp.int32),
            pltpu.VMEM((CHUNK, LANES), jnp.float32),
            pltpu.VMEM_SHARED((NACC, LANES), jnp.float32),
            pltpu.SemaphoreType.DMA,
        ],
        compiler_params=pltpu.CompilerParams(use_tc_tiling_on_sc=False),
    )
    def degree(dst_hbm, out_hbm, dst_v, buf, acc_sh, sem):
        cid = lax.axis_index("c")
        sid = lax.axis_index("s")
        wid = cid * NS + sid
        pltpu.sync_copy(dst_hbm.at[pl.ds(wid * CPT, CPT)], dst_v)
        _zero_buf(buf, LANES)
        for c in range(RPT // CHUNK):
            pltpu.sync_copy(buf, acc_sh.at[pl.ds(sid * RPT + c * CHUNK, CHUNK)])
        plsc.subcore_barrier()
        _fill_ones(buf)

        @pl.loop(0, CPT // 8)
        def _(g):
            for k in range(8):
                pltpu.async_copy(buf, acc_sh.at[dst_v.at[g * 8 + k]], sem,
                                 add=True)
            for k in range(8):
                pltpu.make_async_copy(
                    buf, acc_sh.at[dst_v.at[g * 8 + k]], sem).wait()

        plsc.subcore_barrier()
        _copy_out_stripe(acc_sh, buf, out_hbm, cid, sid, LANES)

    return degree


@functools.cache
def _make_sc_agg_deep(width, K=2):
    RPS = N // NS
    SG = 8
    NV = CPT // (2 * SG)

    @functools.partial(
        pl.kernel,
        out_type=jax.ShapeDtypeStruct((NC, NACC, width), jnp.float32),
        mesh=plsc.VectorSubcoreMesh(core_axis_name="c", subcore_axis_name="s"),
        scratch_types=[
            pltpu.VMEM((2, SG, CHUNK), jnp.int32),
            pltpu.VMEM((2, SG, CHUNK), jnp.int32),
            pltpu.VMEM((2 * K, CHUNK, width), jnp.float32),
            pltpu.VMEM_SHARED((NACC, width), jnp.float32),
            pltpu.VMEM_SHARED((N, width), jnp.float32),
            pltpu.SemaphoreType.DMA,
            pltpu.SemaphoreType.DMA,
            pltpu.SemaphoreType.DMA,
            pltpu.SemaphoreType.DMA,
            pltpu.SemaphoreType.DMA,
            pltpu.SemaphoreType.DMA,
        ],
        compiler_params=pltpu.CompilerParams(use_tc_tiling_on_sc=False),
    )
    def agg(p_hbm, src_hbm, dst_hbm, out_hbm, src_v, dst_v, bufs, acc_sh,
            tab_sh, gsem0, gsem1, ssem0, ssem1, isem0, isem1):
        cid = lax.axis_index("c")
        sid = lax.axis_index("s")
        wid = cid * NS + sid
        base0 = wid * CPT
        gsem = (gsem0, gsem1)
        ssem = (ssem0, ssem1)
        isem = (isem0, isem1)
        GPI = 2 * SG // K

        def fire_idx(sg_base, ib, do_wait=False):
            cp_s = pltpu.async_copy(src_hbm.at[pl.ds(sg_base, SG)],
                                    src_v.at[ib], isem[ib])
            cp_d = pltpu.async_copy(dst_hbm.at[pl.ds(sg_base, SG)],
                                    dst_v.at[ib], isem[ib])
            if do_wait:
                cp_s.wait()
                cp_d.wait()

        def drain_idx(sg_base, ib):
            pltpu.make_async_copy(src_hbm.at[pl.ds(sg_base, SG)],
                                  src_v.at[ib], isem[ib]).wait()
            pltpu.make_async_copy(dst_hbm.at[pl.ds(sg_base, SG)],
                                  dst_v.at[ib], isem[ib]).wait()

        def fire_gathers(gi, sem_i):
            for k in range(K):
                r = (gi * K + k) % (2 * SG)
                pltpu.async_copy(tab_sh.at[src_v.at[r // SG, r % SG]],
                                 bufs.at[(gi % 2) * K + k], gsem[sem_i])

        def drain_gathers(gi, sem_i):
            for k in range(K):
                r = (gi * K + k) % (2 * SG)
                pltpu.make_async_copy(tab_sh.at[src_v.at[r // SG, r % SG]],
                                      bufs.at[(gi % 2) * K + k],
                                      gsem[sem_i]).wait()

        def fire_scatters(gi, sem_i):
            for k in range(K):
                r = (gi * K + k) % (2 * SG)
                pltpu.async_copy(bufs.at[(gi % 2) * K + k],
                                 acc_sh.at[dst_v.at[r // SG, r % SG]],
                                 ssem[sem_i], add=True)

        def drain_scatters(gi, sem_i):
            for k in range(K):
                r = (gi * K + k) % (2 * SG)
                pltpu.make_async_copy(bufs.at[(gi % 2) * K + k],
                                      acc_sh.at[dst_v.at[r // SG, r % SG]],
                                      ssem[sem_i]).wait()

        pltpu.sync_copy(p_hbm.at[pl.ds(sid * RPS, RPS)],
                        tab_sh.at[pl.ds(sid * RPS, RPS)])
        _zero_buf(bufs.at[0], width)
        for c in range(RPT // CHUNK):
            pltpu.sync_copy(bufs.at[0],
                            acc_sh.at[pl.ds(sid * RPT + c * CHUNK, CHUNK)])
        plsc.subcore_barrier()

        fire_idx(base0, 0, do_wait=True)
        fire_idx(base0 + SG, 1)

        @pl.loop(0, NV)
        def _(v):
            fire_gathers(0, 0)
            for gi in range(GPI):
                drain_gathers(gi, gi % 2)
                fire_scatters(gi, gi % 2)
                if gi == SG // K - 1:
                    drain_idx(base0 + v * 2 * SG + SG, 1)
                if gi >= 1:
                    drain_scatters(gi - 1, (gi - 1) % 2)
                if gi < GPI - 1:
                    fire_gathers(gi + 1, (gi + 1) % 2)
            drain_scatters(GPI - 1, (GPI - 1) % 2)

            @pl.when(v < NV - 1)
            def _():
                nxt = base0 + (v + 1) * 2 * SG
                fire_idx(nxt, 0)
                fire_idx(nxt + SG, 1)
                drain_idx(nxt, 0)

        plsc.subcore_barrier()
        _copy_out_stripe(acc_sh, bufs.at[0], out_hbm, cid, sid, width)

    return agg


@functools.cache
def _make_sc_agg(width, c0=CPT, c1=CPT, K=4):
    cmax = max(c0, c1)
    RPS = N // NS

    @functools.partial(
        pl.kernel,
        out_type=jax.ShapeDtypeStruct((NC, NACC, width), jnp.float32),
        mesh=plsc.VectorSubcoreMesh(core_axis_name="c", subcore_axis_name="s"),
        scratch_types=[
            pltpu.VMEM((cmax, CHUNK), jnp.int32),
            pltpu.VMEM((cmax, CHUNK), jnp.int32),
            pltpu.VMEM((2 * K, CHUNK, width), jnp.float32),
            pltpu.VMEM_SHARED((NACC, width), jnp.float32),
            pltpu.VMEM_SHARED((N, width), jnp.float32),
            pltpu.SemaphoreType.DMA,
            pltpu.SemaphoreType.DMA,
            pltpu.SemaphoreType.DMA,
            pltpu.SemaphoreType.DMA,
        ],
        compiler_params=pltpu.CompilerParams(use_tc_tiling_on_sc=False),
    )
    def agg(p_hbm, src_hbm, dst_hbm, out_hbm, src_v, dst_v, bufs, acc_sh,
            tab_sh, gsem0, gsem1, ssem0, ssem1):
        cid = lax.axis_index("c")
        sid = lax.axis_index("s")

        def fire_gathers(base, bank, sem):
            for k in range(K):
                pltpu.async_copy(tab_sh.at[src_v.at[base + k]],
                                 bufs.at[bank * K + k], sem)

        def drain_gathers(base, bank, sem):
            for k in range(K):
                pltpu.make_async_copy(tab_sh.at[src_v.at[base + k]],
                                      bufs.at[bank * K + k], sem).wait()

        def fire_scatters(base, bank, sem):
            for k in range(K):
                pltpu.async_copy(bufs.at[bank * K + k],
                                 acc_sh.at[dst_v.at[base + k]], sem, add=True)

        def drain_scatters(base, bank, sem):
            for k in range(K):
                pltpu.make_async_copy(bufs.at[bank * K + k],
                                      acc_sh.at[dst_v.at[base + k]],
                                      sem).wait()

        def run(hbm_base, nchunks):
            pltpu.sync_copy(src_hbm.at[pl.ds(hbm_base, nchunks)],
                            src_v.at[pl.ds(0, nchunks)])
            pltpu.sync_copy(dst_hbm.at[pl.ds(hbm_base, nchunks)],
                            dst_v.at[pl.ds(0, nchunks)])
            ng = nchunks // K
            fire_gathers(0, 0, gsem0)

            @pl.loop(0, ng // 2)
            def _(t):
                a = t * (2 * K)
                b = a + K
                drain_gathers(a, 0, gsem0)
                fire_scatters(a, 0, ssem0)

                @pl.when(t > 0)
                def _():
                    drain_scatters(a - K, 1, ssem1)

                fire_gathers(b, 1, gsem1)
                drain_gathers(b, 1, gsem1)
                fire_scatters(b, 1, ssem1)
                drain_scatters(a, 0, ssem0)

                @pl.when(t < ng // 2 - 1)
                def _():
                    fire_gathers(b + K, 0, gsem0)

            drain_scatters(nchunks - K, 1, ssem1)

        pltpu.sync_copy(p_hbm.at[pl.ds(sid * RPS, RPS)],
                        tab_sh.at[pl.ds(sid * RPS, RPS)])
        _zero_buf(bufs.at[0], width)
        for c in range(RPT // CHUNK):
            pltpu.sync_copy(bufs.at[0],
                            acc_sh.at[pl.ds(sid * RPT + c * CHUNK, CHUNK)])
        plsc.subcore_barrier()

        if c0 == c1:
            run(cid * NS * c0 + sid * c0, c0)
        else:
            @pl.when(cid == 0)
            def _():
                run(sid * c0, c0)

            if c1 > 0:
                @pl.when(cid == 1)
                def _():
                    run(NS * c0 + sid * c1, c1)

        plsc.subcore_barrier()
        _copy_out_stripe(acc_sh, bufs.at[0], out_hbm, cid, sid, width)

    return agg


def _tc_prep_body(deg0, deg1, x_ref, w_ref, p_ref):
    dis = lax.rsqrt(deg0[...] + deg1[...] + 1.0)
    h = jnp.dot(x_ref[...], w_ref[...], preferred_element_type=jnp.float32)
    p_ref[...] = dis * h


def _tc_mid_body(deg0, deg1, acc_ref, p_ref, w_ref, b_ref, out_ref):
    dis = lax.rsqrt(deg0[...] + deg1[...] + 1.0)
    s = acc_ref[0, :N, :] + acc_ref[1, :N, :] + p_ref[...]
    out = jax.nn.relu(dis * s + b_ref[...])
    h = jnp.dot(out, w_ref[...], preferred_element_type=jnp.float32)
    out_ref[...] = dis * h


def _tc_final_body(deg0, deg1, acc_ref, q_ref, wmu_ref, bmu_ref, wlv_ref,
                   blv_ref, mu_ref, lv_ref):
    dis = lax.rsqrt(deg0[...] + deg1[...] + 1.0)
    t = dis * (acc_ref[0, :N, :] + acc_ref[1, :N, :] + q_ref[...])
    mu_ref[...] = (
        jnp.dot(t, wmu_ref[...], preferred_element_type=jnp.float32)
        + bmu_ref[...]
    )
    lv_ref[...] = (
        jnp.dot(t, wlv_ref[...], preferred_element_type=jnp.float32)
        + blv_ref[...]
    )


def _pad2(a, rows, cols):
    out = jnp.zeros((rows, cols), jnp.float32)
    return out.at[: a.shape[0], : a.shape[1]].set(a)


def kernel(x, edge_index, W1, b1, W2, b2, W3, b3, Wmu, bmu, Wlv, blv):
    ei = edge_index.astype(jnp.int32)
    src = jnp.concatenate([ei[0], jnp.zeros((EP - E,), jnp.int32)])
    dst = jnp.concatenate([ei[1], jnp.full((EP - E,), N, jnp.int32)])
    src2d = src.reshape(EP // CHUNK, CHUNK)
    dst2d = dst.reshape(EP // CHUNK, CHUNK)

    degs = _make_sc_degree()(dst2d)
    _sc_agg64 = _make_sc_agg_deep(64, 2)
    _sc_agg16 = _make_sc_agg(16, CPT, CPT, 4)
    deg0 = degs[0, :N, 0:1]
    deg1 = degs[1, :N, 0:1]

    p1 = pl.pallas_call(
        _tc_prep_body,
        out_shape=jax.ShapeDtypeStruct((N, 64), jnp.float32),
    )(deg0, deg1, x, W1)
    acc1 = _sc_agg64(p1, src2d, dst2d)

    w2p = _pad2(W2, 64, 16)
    p2 = pl.pallas_call(
        _tc_mid_body,
        out_shape=jax.ShapeDtypeStruct((N, 16), jnp.float32),
    )(deg0, deg1, acc1, p1, w2p, b1.reshape(1, 64))
    acc2 = _sc_agg16(p2, src2d, dst2d)

    w3p = _pad2(W3, 16, 16)
    b2p = _pad2(b2.reshape(1, 8), 1, 16)
    p3 = pl.pallas_call(
        _tc_mid_body,
        out_shape=jax.ShapeDtypeStruct((N, 16), jnp.float32),
    )(deg0, deg1, acc2, p2, w3p, b2p)
    acc3 = _sc_agg16(p3, src2d, dst2d)

    eye = jnp.eye(16, dtype=jnp.float32)
    b3p = _pad2(b3.reshape(1, 4), 1, 16)
    q = pl.pallas_call(
        _tc_mid_body,
        out_shape=jax.ShapeDtypeStruct((N, 16), jnp.float32),
    )(deg0, deg1, acc3, p3, eye, b3p)
    acc4 = _sc_agg16(q, src2d, dst2d)

    wmup = _pad2(Wmu, 16, 2)
    wlvp = _pad2(Wlv, 16, 2)
    mu, lv = pl.pallas_call(
        _tc_final_body,
        out_shape=[
            jax.ShapeDtypeStruct((N, 2), jnp.float32),
            jax.ShapeDtypeStruct((N, 2), jnp.float32),
        ],
    )(deg0, deg1, acc4, q, wmup, bmu.reshape(1, 2),
      wlvp, blv.reshape(1, 2))
    return (mu, lv)


# speedup vs baseline: 1.0153x; 1.0153x over previous
import functools

import jax
import jax.numpy as jnp
from jax import lax
from jax.experimental import pallas as pl
from jax.experimental.pallas import tpu as pltpu
from jax.experimental.pallas import tpu_sc as plsc

N = 10000
E = 320000
NC = 2
NS = 16
LANES = 16
NW = NC * NS
CHUNK = 128
CPT = 80
EPT = CHUNK * CPT
EP = EPT * NW
RPT = 640
NACC = RPT * NS

def _zero_buf(buf, width):
    @pl.loop(0, CHUNK)
    def _(r):
        for c in range(width // LANES):
            buf[r, pl.ds(c * LANES, LANES)] = jnp.zeros((LANES,), jnp.float32)


def _fill_ones(buf):
    @pl.loop(0, CHUNK)
    def _(r):
        buf[r, :] = jnp.full((LANES,), 1.0, jnp.float32)


def _copy_out_stripe(acc_sh, buf, out_hbm, cid, sid, width):
    for c in range(RPT // CHUNK):
        row = sid * RPT + c * CHUNK
        pltpu.sync_copy(acc_sh.at[pl.ds(row, CHUNK)], buf)
        pltpu.sync_copy(buf, out_hbm.at[cid, pl.ds(row, CHUNK)])


@functools.cache
def _make_sc_degree():
    @functools.partial(
        pl.kernel,
        out_type=jax.ShapeDtypeStruct((NC, NACC, LANES), jnp.float32),
        mesh=plsc.VectorSubcoreMesh(core_axis_name="c", subcore_axis_name="s"),
        scratch_types=[
            pltpu.VMEM((CPT, CHUNK), jnp.int32),
            pltpu.VMEM((CHUNK, LANES), jnp.float32),
            pltpu.VMEM_SHARED((NACC, LANES), jnp.float32),
            pltpu.SemaphoreType.DMA,
        ],
        compiler_params=pltpu.CompilerParams(use_tc_tiling_on_sc=False),
    )
    def degree(dst_hbm, out_hbm, dst_v, buf, acc_sh, sem):
        cid = lax.axis_index("c")
        sid = lax.axis_index("s")
        wid = cid * NS + sid
        pltpu.sync_copy(dst_hbm.at[pl.ds(wid * CPT, CPT)], dst_v)
        _zero_buf(buf, LANES)
        for c in range(RPT // CHUNK):
            pltpu.sync_copy(buf, acc_sh.at[pl.ds(sid * RPT + c * CHUNK, CHUNK)])
        plsc.subcore_barrier()
        _fill_ones(buf)

        @pl.loop(0, CPT // 8)
        def _(g):
            for k in range(8):
                pltpu.async_copy(buf, acc_sh.at[dst_v.at[g * 8 + k]], sem,
                                 add=True)
            for k in range(8):
                pltpu.make_async_copy(
                    buf, acc_sh.at[dst_v.at[g * 8 + k]], sem).wait()

        plsc.subcore_barrier()
        _copy_out_stripe(acc_sh, buf, out_hbm, cid, sid, LANES)

    return degree


@functools.cache
def _make_sc_agg_deep(width, K=2):
    RPS = N // NS
    SG = 8
    NV = CPT // (2 * SG)

    @functools.partial(
        pl.kernel,
        out_type=jax.ShapeDtypeStruct((NC, NACC, width), jnp.float32),
        mesh=plsc.VectorSubcoreMesh(core_axis_name="c", subcore_axis_name="s"),
        scratch_types=[
            pltpu.VMEM((2, SG, CHUNK), jnp.int32),
            pltpu.VMEM((2, SG, CHUNK), jnp.int32),
            pltpu.VMEM((2 * K, CHUNK, width), jnp.float32),
            pltpu.VMEM_SHARED((NACC, width), jnp.float32),
            pltpu.VMEM_SHARED((N, width), jnp.float32),
            pltpu.SemaphoreType.DMA,
            pltpu.SemaphoreType.DMA,
            pltpu.SemaphoreType.DMA,
            pltpu.SemaphoreType.DMA,
            pltpu.SemaphoreType.DMA,
            pltpu.SemaphoreType.DMA,
        ],
        compiler_params=pltpu.CompilerParams(use_tc_tiling_on_sc=False),
    )
    def agg(p_hbm, src_hbm, dst_hbm, out_hbm, src_v, dst_v, bufs, acc_sh,
            tab_sh, gsem0, gsem1, ssem0, ssem1, isem0, isem1):
        cid = lax.axis_index("c")
        sid = lax.axis_index("s")
        wid = cid * NS + sid
        base0 = wid * CPT
        gsem = (gsem0, gsem1)
        ssem = (ssem0, ssem1)
        isem = (isem0, isem1)
        GPI = 2 * SG // K

        def fire_idx(sg_base, ib, do_wait=False):
            cp_s = pltpu.async_copy(src_hbm.at[pl.ds(sg_base, SG)],
                                    src_v.at[ib], isem[ib])
            cp_d = pltpu.async_copy(dst_hbm.at[pl.ds(sg_base, SG)],
                                    dst_v.at[ib], isem[ib])
            if do_wait:
                cp_s.wait()
                cp_d.wait()

        def drain_idx(sg_base, ib):
            pltpu.make_async_copy(src_hbm.at[pl.ds(sg_base, SG)],
                                  src_v.at[ib], isem[ib]).wait()
            pltpu.make_async_copy(dst_hbm.at[pl.ds(sg_base, SG)],
                                  dst_v.at[ib], isem[ib]).wait()

        def fire_gathers(gi, sem_i):
            for k in range(K):
                r = (gi * K + k) % (2 * SG)
                pltpu.async_copy(tab_sh.at[src_v.at[r // SG, r % SG]],
                                 bufs.at[(gi % 2) * K + k], gsem[sem_i])

        def drain_gathers(gi, sem_i):
            for k in range(K):
                r = (gi * K + k) % (2 * SG)
                pltpu.make_async_copy(tab_sh.at[src_v.at[r // SG, r % SG]],
                                      bufs.at[(gi % 2) * K + k],
                                      gsem[sem_i]).wait()

        def fire_scatters(gi, sem_i):
            for k in range(K):
                r = (gi * K + k) % (2 * SG)
                pltpu.async_copy(bufs.at[(gi % 2) * K + k],
                                 acc_sh.at[dst_v.at[r // SG, r % SG]],
                                 ssem[sem_i], add=True)

        def drain_scatters(gi, sem_i):
            for k in range(K):
                r = (gi * K + k) % (2 * SG)
                pltpu.make_async_copy(bufs.at[(gi % 2) * K + k],
                                      acc_sh.at[dst_v.at[r // SG, r % SG]],
                                      ssem[sem_i]).wait()

        pltpu.sync_copy(p_hbm.at[pl.ds(sid * RPS, RPS)],
                        tab_sh.at[pl.ds(sid * RPS, RPS)])
        _zero_buf(bufs.at[0], width)
        for c in range(RPT // CHUNK):
            pltpu.sync_copy(bufs.at[0],
                            acc_sh.at[pl.ds(sid * RPT + c * CHUNK, CHUNK)])
        plsc.subcore_barrier()

        fire_idx(base0, 0, do_wait=True)
        fire_idx(base0 + SG, 1)

        @pl.loop(0, NV)
        def _(v):
            fire_gathers(0, 0)
            for gi in range(GPI):
                drain_gathers(gi, gi % 2)
                fire_scatters(gi, gi % 2)
                if gi == SG // K - 1:
                    drain_idx(base0 + v * 2 * SG + SG, 1)
                if gi >= 1:
                    drain_scatters(gi - 1, (gi - 1) % 2)
                if gi < GPI - 1:
                    fire_gathers(gi + 1, (gi + 1) % 2)
            drain_scatters(GPI - 1, (GPI - 1) % 2)

            @pl.when(v < NV - 1)
            def _():
                nxt = base0 + (v + 1) * 2 * SG
                fire_idx(nxt, 0)
                fire_idx(nxt + SG, 1)
                drain_idx(nxt, 0)

        plsc.subcore_barrier()
        _copy_out_stripe(acc_sh, bufs.at[0], out_hbm, cid, sid, width)

    return agg


@functools.cache
def _make_sc_agg(width, c0=CPT, c1=CPT, K=4):
    cmax = max(c0, c1)
    RPS = N // NS

    @functools.partial(
        pl.kernel,
        out_type=jax.ShapeDtypeStruct((NC, NACC, width), jnp.float32),
        mesh=plsc.VectorSubcoreMesh(core_axis_name="c", subcore_axis_name="s"),
        scratch_types=[
            pltpu.VMEM((cmax, CHUNK), jnp.int32),
            pltpu.VMEM((cmax, CHUNK), jnp.int32),
            pltpu.VMEM((2 * K, CHUNK, width), jnp.float32),
            pltpu.VMEM_SHARED((NACC, width), jnp.float32),
            pltpu.VMEM_SHARED((N, width), jnp.float32),
            pltpu.SemaphoreType.DMA,
            pltpu.SemaphoreType.DMA,
            pltpu.SemaphoreType.DMA,
            pltpu.SemaphoreType.DMA,
        ],
        compiler_params=pltpu.CompilerParams(use_tc_tiling_on_sc=False),
    )
    def agg(p_hbm, src_hbm, dst_hbm, out_hbm, src_v, dst_v, bufs, acc_sh,
            tab_sh, gsem0, gsem1, ssem0, ssem1):
        cid = lax.axis_index("c")
        sid = lax.axis_index("s")

        def fire_gathers(base, bank, sem):
            for k in range(K):
                pltpu.async_copy(tab_sh.at[src_v.at[base + k]],
                                 bufs.at[bank * K + k], sem)

        def drain_gathers(base, bank, sem):
            for k in range(K):
                pltpu.make_async_copy(tab_sh.at[src_v.at[base + k]],
                                      bufs.at[bank * K + k], sem).wait()

        def fire_scatters(base, bank, sem):
            for k in range(K):
                pltpu.async_copy(bufs.at[bank * K + k],
                                 acc_sh.at[dst_v.at[base + k]], sem, add=True)

        def drain_scatters(base, bank, sem):
            for k in range(K):
                pltpu.make_async_copy(bufs.at[bank * K + k],
                                      acc_sh.at[dst_v.at[base + k]],
                                      sem).wait()

        def run(hbm_base, nchunks):
            pltpu.sync_copy(src_hbm.at[pl.ds(hbm_base, nchunks)],
                            src_v.at[pl.ds(0, nchunks)])
            pltpu.sync_copy(dst_hbm.at[pl.ds(hbm_base, nchunks)],
                            dst_v.at[pl.ds(0, nchunks)])
            ng = nchunks // K
            fire_gathers(0, 0, gsem0)

            @pl.loop(0, ng // 2)
            def _(t):
                a = t * (2 * K)
                b = a + K
                drain_gathers(a, 0, gsem0)
                fire_scatters(a, 0, ssem0)

                @pl.when(t > 0)
                def _():
                    drain_scatters(a - K, 1, ssem1)

                fire_gathers(b, 1, gsem1)
                drain_gathers(b, 1, gsem1)
                fire_scatters(b, 1, ssem1)
                drain_scatters(a, 0, ssem0)

                @pl.when(t < ng // 2 - 1)
                def _():
                    fire_gathers(b + K, 0, gsem0)

            drain_scatters(nchunks - K, 1, ssem1)

        pltpu.sync_copy(p_hbm.at[pl.ds(sid * RPS, RPS)],
                        tab_sh.at[pl.ds(sid * RPS, RPS)])
        _zero_buf(bufs.at[0], width)
        for c in range(RPT // CHUNK):
            pltpu.sync_copy(bufs.at[0],
                            acc_sh.at[pl.ds(sid * RPT + c * CHUNK, CHUNK)])
        plsc.subcore_barrier()

        if c0 == c1:
            run(cid * NS * c0 + sid * c0, c0)
        else:
            @pl.when(cid == 0)
            def _():
                run(sid * c0, c0)

            if c1 > 0:
                @pl.when(cid == 1)
                def _():
                    run(NS * c0 + sid * c1, c1)

        plsc.subcore_barrier()
        _copy_out_stripe(acc_sh, bufs.at[0], out_hbm, cid, sid, width)

    return agg


def _tc_matmul_body(x_ref, w_ref, h_ref):
    h_ref[...] = jnp.dot(x_ref[...], w_ref[...],
                         preferred_element_type=jnp.float32)


def _tc_scale_body(deg0, deg1, h_ref, p_ref):
    dis = lax.rsqrt(deg0[...] + deg1[...] + 1.0)
    p_ref[...] = dis * h_ref[...]


def _tc_mid_body(deg0, deg1, acc_ref, p_ref, w_ref, b_ref, out_ref):
    dis = lax.rsqrt(deg0[...] + deg1[...] + 1.0)
    s = acc_ref[0, :N, :] + acc_ref[1, :N, :] + p_ref[...]
    out = jax.nn.relu(dis * s + b_ref[...])
    h = jnp.dot(out, w_ref[...], preferred_element_type=jnp.float32)
    out_ref[...] = dis * h


def _tc_final_body(deg0, deg1, acc_ref, q_ref, wmu_ref, bmu_ref, wlv_ref,
                   blv_ref, mu_ref, lv_ref):
    dis = lax.rsqrt(deg0[...] + deg1[...] + 1.0)
    t = dis * (acc_ref[0, :N, :] + acc_ref[1, :N, :] + q_ref[...])
    mu_ref[...] = (
        jnp.dot(t, wmu_ref[...], preferred_element_type=jnp.float32)
        + bmu_ref[...]
    )
    lv_ref[...] = (
        jnp.dot(t, wlv_ref[...], preferred_element_type=jnp.float32)
        + blv_ref[...]
    )


def _pad2(a, rows, cols):
    out = jnp.zeros((rows, cols), jnp.float32)
    return out.at[: a.shape[0], : a.shape[1]].set(a)


def kernel(x, edge_index, W1, b1, W2, b2, W3, b3, Wmu, bmu, Wlv, blv):
    ei = edge_index.astype(jnp.int32)
    src = jnp.concatenate([ei[0], jnp.zeros((EP - E,), jnp.int32)])
    dst = jnp.concatenate([ei[1], jnp.full((EP - E,), N, jnp.int32)])
    src2d = src.reshape(EP // CHUNK, CHUNK)
    dst2d = dst.reshape(EP // CHUNK, CHUNK)

    degs = _make_sc_degree()(dst2d)
    _sc_agg64 = _make_sc_agg(64, 88, 72, 1)
    _sc_agg16 = _make_sc_agg(16, 88, 72, 4)
    deg0 = degs[0, :N, 0:1]
    deg1 = degs[1, :N, 0:1]

    h1 = pl.pallas_call(
        _tc_matmul_body,
        out_shape=jax.ShapeDtypeStruct((N, 64), jnp.float32),
    )(x, W1)
    p1 = pl.pallas_call(
        _tc_scale_body,
        out_shape=jax.ShapeDtypeStruct((N, 64), jnp.float32),
    )(deg0, deg1, h1)
    acc1 = _sc_agg64(p1, src2d, dst2d)

    w2p = _pad2(W2, 64, 16)
    p2 = pl.pallas_call(
        _tc_mid_body,
        out_shape=jax.ShapeDtypeStruct((N, 16), jnp.float32),
    )(deg0, deg1, acc1, p1, w2p, b1.reshape(1, 64))
    acc2 = _sc_agg16(p2, src2d, dst2d)

    w3p = _pad2(W3, 16, 16)
    b2p = _pad2(b2.reshape(1, 8), 1, 16)
    p3 = pl.pallas_call(
        _tc_mid_body,
        out_shape=jax.ShapeDtypeStruct((N, 16), jnp.float32),
    )(deg0, deg1, acc2, p2, w3p, b2p)
    acc3 = _sc_agg16(p3, src2d, dst2d)

    eye = jnp.eye(16, dtype=jnp.float32)
    b3p = _pad2(b3.reshape(1, 4), 1, 16)
    q = pl.pallas_call(
        _tc_mid_body,
        out_shape=jax.ShapeDtypeStruct((N, 16), jnp.float32),
    )(deg0, deg1, acc3, p3, eye, b3p)
    acc4 = _sc_agg16(q, src2d, dst2d)

    wmup = _pad2(Wmu, 16, 2)
    wlvp = _pad2(Wlv, 16, 2)
    mu, lv = pl.pallas_call(
        _tc_final_body,
        out_shape=[
            jax.ShapeDtypeStruct((N, 2), jnp.float32),
            jax.ShapeDtypeStruct((N, 2), jnp.float32),
        ],
    )(deg0, deg1, acc4, q, wmup, bmu.reshape(1, 2),
      wlvp, blv.reshape(1, 2))
    return (mu, lv)


# speedup vs baseline: 1.0369x; 1.0212x over previous
import functools

import jax
import jax.numpy as jnp
from jax import lax
from jax.experimental import pallas as pl
from jax.experimental.pallas import tpu as pltpu
from jax.experimental.pallas import tpu_sc as plsc

N = 10000
E = 320000
NC = 2
NS = 16
LANES = 16
NW = NC * NS
CHUNK = 128
CPT = 80
EPT = CHUNK * CPT
EP = EPT * NW
RPT = 640
NACC = RPT * NS

def _zero_buf(buf, width):
    @pl.loop(0, CHUNK)
    def _(r):
        for c in range(width // LANES):
            buf[r, pl.ds(c * LANES, LANES)] = jnp.zeros((LANES,), jnp.float32)


def _fill_ones(buf):
    @pl.loop(0, CHUNK)
    def _(r):
        buf[r, :] = jnp.full((LANES,), 1.0, jnp.float32)


def _copy_out_stripe(acc_sh, buf, out_hbm, cid, sid, width):
    for c in range(RPT // CHUNK):
        row = sid * RPT + c * CHUNK
        pltpu.sync_copy(acc_sh.at[pl.ds(row, CHUNK)], buf)
        pltpu.sync_copy(buf, out_hbm.at[cid, pl.ds(row, CHUNK)])


DEGR = 640


@functools.cache
def _make_sc_degree_hist():
    STR = DEGR // NS

    @functools.partial(
        pl.kernel,
        out_type=jax.ShapeDtypeStruct((NC, DEGR, LANES), jnp.float32),
        mesh=plsc.VectorSubcoreMesh(core_axis_name="c", subcore_axis_name="s"),
        scratch_types=[
            pltpu.VMEM((CPT, CHUNK), jnp.int32),
            pltpu.VMEM((DEGR, LANES), jnp.float32),
            pltpu.VMEM((DEGR // CHUNK, CHUNK), jnp.int32),
            pltpu.VMEM_SHARED((DEGR, LANES), jnp.float32),
        ],
        compiler_params=pltpu.CompilerParams(use_tc_tiling_on_sc=False,
                                             needs_layout_passes=False),
    )
    def degree(dst_hbm, iota_hbm, out_hbm, dst_v, deg_v, iota_v, acc_sh):
        cid = lax.axis_index("c")
        sid = lax.axis_index("s")
        wid = cid * NS + sid
        pltpu.sync_copy(dst_hbm.at[pl.ds(wid * CPT, CPT)], dst_v)
        pltpu.sync_copy(iota_hbm, iota_v)

        @pl.loop(0, DEGR)
        def _(r):
            deg_v[r, :] = jnp.zeros((LANES,), jnp.float32)

        pltpu.sync_copy(deg_v.at[pl.ds(0, STR)],
                        acc_sh.at[pl.ds(sid * STR, STR)])
        plsc.subcore_barrier()

        ones = jnp.full((LANES,), 1.0, jnp.float32)

        @pl.loop(0, CPT)
        def _(j):
            for k in range(CHUNK // LANES):
                idx = dst_v[j, pl.ds(k * LANES, LANES)]
                row = lax.shift_right_logical(idx, 4)
                col = lax.bitwise_and(idx, 15)
                plsc.addupdate_scatter(deg_v, [row, col], ones)

        for c in range(DEGR // CHUNK):
            pltpu.sync_copy(deg_v.at[pl.ds(c * CHUNK, CHUNK)],
                            acc_sh.at[iota_v.at[c]], add=True)
        plsc.subcore_barrier()
        pltpu.sync_copy(acc_sh.at[pl.ds(sid * STR, STR)],
                        deg_v.at[pl.ds(0, STR)])
        pltpu.sync_copy(deg_v.at[pl.ds(0, STR)],
                        out_hbm.at[cid, pl.ds(sid * STR, STR)])

    return degree


@functools.cache
def _make_sc_degree():
    @functools.partial(
        pl.kernel,
        out_type=jax.ShapeDtypeStruct((NC, NACC, LANES), jnp.float32),
        mesh=plsc.VectorSubcoreMesh(core_axis_name="c", subcore_axis_name="s"),
        scratch_types=[
            pltpu.VMEM((CPT, CHUNK), jnp.int32),
            pltpu.VMEM((CHUNK, LANES), jnp.float32),
            pltpu.VMEM_SHARED((NACC, LANES), jnp.float32),
            pltpu.SemaphoreType.DMA,
        ],
        compiler_params=pltpu.CompilerParams(use_tc_tiling_on_sc=False),
    )
    def degree(dst_hbm, out_hbm, dst_v, buf, acc_sh, sem):
        cid = lax.axis_index("c")
        sid = lax.axis_index("s")
        wid = cid * NS + sid
        pltpu.sync_copy(dst_hbm.at[pl.ds(wid * CPT, CPT)], dst_v)
        _zero_buf(buf, LANES)
        for c in range(RPT // CHUNK):
            pltpu.sync_copy(buf, acc_sh.at[pl.ds(sid * RPT + c * CHUNK, CHUNK)])
        plsc.subcore_barrier()
        _fill_ones(buf)

        @pl.loop(0, CPT // 8)
        def _(g):
            for k in range(8):
                pltpu.async_copy(buf, acc_sh.at[dst_v.at[g * 8 + k]], sem,
                                 add=True)
            for k in range(8):
                pltpu.make_async_copy(
                    buf, acc_sh.at[dst_v.at[g * 8 + k]], sem).wait()

        plsc.subcore_barrier()
        _copy_out_stripe(acc_sh, buf, out_hbm, cid, sid, LANES)

    return degree


@functools.cache
def _make_sc_agg_deep(width, K=2):
    RPS = N // NS
    SG = 8
    NV = CPT // (2 * SG)

    @functools.partial(
        pl.kernel,
        out_type=jax.ShapeDtypeStruct((NC, NACC, width), jnp.float32),
        mesh=plsc.VectorSubcoreMesh(core_axis_name="c", subcore_axis_name="s"),
        scratch_types=[
            pltpu.VMEM((2, SG, CHUNK), jnp.int32),
            pltpu.VMEM((2, SG, CHUNK), jnp.int32),
            pltpu.VMEM((2 * K, CHUNK, width), jnp.float32),
            pltpu.VMEM_SHARED((NACC, width), jnp.float32),
            pltpu.VMEM_SHARED((N, width), jnp.float32),
            pltpu.SemaphoreType.DMA,
            pltpu.SemaphoreType.DMA,
            pltpu.SemaphoreType.DMA,
            pltpu.SemaphoreType.DMA,
            pltpu.SemaphoreType.DMA,
            pltpu.SemaphoreType.DMA,
        ],
        compiler_params=pltpu.CompilerParams(use_tc_tiling_on_sc=False),
    )
    def agg(p_hbm, src_hbm, dst_hbm, out_hbm, src_v, dst_v, bufs, acc_sh,
            tab_sh, gsem0, gsem1, ssem0, ssem1, isem0, isem1):
        cid = lax.axis_index("c")
        sid = lax.axis_index("s")
        wid = cid * NS + sid
        base0 = wid * CPT
        gsem = (gsem0, gsem1)
        ssem = (ssem0, ssem1)
        isem = (isem0, isem1)
        GPI = 2 * SG // K

        def fire_idx(sg_base, ib, do_wait=False):
            cp_s = pltpu.async_copy(src_hbm.at[pl.ds(sg_base, SG)],
                                    src_v.at[ib], isem[ib])
            cp_d = pltpu.async_copy(dst_hbm.at[pl.ds(sg_base, SG)],
                                    dst_v.at[ib], isem[ib])
            if do_wait:
                cp_s.wait()
                cp_d.wait()

        def drain_idx(sg_base, ib):
            pltpu.make_async_copy(src_hbm.at[pl.ds(sg_base, SG)],
                                  src_v.at[ib], isem[ib]).wait()
            pltpu.make_async_copy(dst_hbm.at[pl.ds(sg_base, SG)],
                                  dst_v.at[ib], isem[ib]).wait()

        def fire_gathers(gi, sem_i):
            for k in range(K):
                r = (gi * K + k) % (2 * SG)
                pltpu.async_copy(tab_sh.at[src_v.at[r // SG, r % SG]],
                                 bufs.at[(gi % 2) * K + k], gsem[sem_i])

        def drain_gathers(gi, sem_i):
            for k in range(K):
                r = (gi * K + k) % (2 * SG)
                pltpu.make_async_copy(tab_sh.at[src_v.at[r // SG, r % SG]],
                                      bufs.at[(gi % 2) * K + k],
                                      gsem[sem_i]).wait()

        def fire_scatters(gi, sem_i):
            for k in range(K):
                r = (gi * K + k) % (2 * SG)
                pltpu.async_copy(bufs.at[(gi % 2) * K + k],
                                 acc_sh.at[dst_v.at[r // SG, r % SG]],
                                 ssem[sem_i], add=True)

        def drain_scatters(gi, sem_i):
            for k in range(K):
                r = (gi * K + k) % (2 * SG)
                pltpu.make_async_copy(bufs.at[(gi % 2) * K + k],
                                      acc_sh.at[dst_v.at[r // SG, r % SG]],
                                      ssem[sem_i]).wait()

        pltpu.sync_copy(p_hbm.at[pl.ds(sid * RPS, RPS)],
                        tab_sh.at[pl.ds(sid * RPS, RPS)])
        _zero_buf(bufs.at[0], width)
        for c in range(RPT // CHUNK):
            pltpu.sync_copy(bufs.at[0],
                            acc_sh.at[pl.ds(sid * RPT + c * CHUNK, CHUNK)])
        plsc.subcore_barrier()

        fire_idx(base0, 0, do_wait=True)
        fire_idx(base0 + SG, 1)

        @pl.loop(0, NV)
        def _(v):
            fire_gathers(0, 0)
            for gi in range(GPI):
                drain_gathers(gi, gi % 2)
                fire_scatters(gi, gi % 2)
                if gi == SG // K - 1:
                    drain_idx(base0 + v * 2 * SG + SG, 1)
                if gi >= 1:
                    drain_scatters(gi - 1, (gi - 1) % 2)
                if gi < GPI - 1:
                    fire_gathers(gi + 1, (gi + 1) % 2)
            drain_scatters(GPI - 1, (GPI - 1) % 2)

            @pl.when(v < NV - 1)
            def _():
                nxt = base0 + (v + 1) * 2 * SG
                fire_idx(nxt, 0)
                fire_idx(nxt + SG, 1)
                drain_idx(nxt, 0)

        plsc.subcore_barrier()
        _copy_out_stripe(acc_sh, bufs.at[0], out_hbm, cid, sid, width)

    return agg


@functools.cache
def _make_sc_agg(width, c0=CPT, c1=CPT, K=4):
    cmax = max(c0, c1)
    RPS = N // NS

    @functools.partial(
        pl.kernel,
        out_type=jax.ShapeDtypeStruct((NC, NACC, width), jnp.float32),
        mesh=plsc.VectorSubcoreMesh(core_axis_name="c", subcore_axis_name="s"),
        scratch_types=[
            pltpu.VMEM((cmax, CHUNK), jnp.int32),
            pltpu.VMEM((cmax, CHUNK), jnp.int32),
            pltpu.VMEM((2 * K, CHUNK, width), jnp.float32),
            pltpu.VMEM_SHARED((NACC, width), jnp.float32),
            pltpu.VMEM_SHARED((N, width), jnp.float32),
            pltpu.SemaphoreType.DMA,
            pltpu.SemaphoreType.DMA,
            pltpu.SemaphoreType.DMA,
            pltpu.SemaphoreType.DMA,
        ],
        compiler_params=pltpu.CompilerParams(use_tc_tiling_on_sc=False),
    )
    def agg(p_hbm, src_hbm, dst_hbm, out_hbm, src_v, dst_v, bufs, acc_sh,
            tab_sh, gsem0, gsem1, ssem0, ssem1):
        cid = lax.axis_index("c")
        sid = lax.axis_index("s")

        def fire_gathers(base, bank, sem):
            for k in range(K):
                pltpu.async_copy(tab_sh.at[src_v.at[base + k]],
                                 bufs.at[bank * K + k], sem)

        def drain_gathers(base, bank, sem):
            for k in range(K):
                pltpu.make_async_copy(tab_sh.at[src_v.at[base + k]],
                                      bufs.at[bank * K + k], sem).wait()

        def fire_scatters(base, bank, sem):
            for k in range(K):
                pltpu.async_copy(bufs.at[bank * K + k],
                                 acc_sh.at[dst_v.at[base + k]], sem, add=True)

        def drain_scatters(base, bank, sem):
            for k in range(K):
                pltpu.make_async_copy(bufs.at[bank * K + k],
                                      acc_sh.at[dst_v.at[base + k]],
                                      sem).wait()

        def run(hbm_base, nchunks):
            pltpu.sync_copy(src_hbm.at[pl.ds(hbm_base, nchunks)],
                            src_v.at[pl.ds(0, nchunks)])
            pltpu.sync_copy(dst_hbm.at[pl.ds(hbm_base, nchunks)],
                            dst_v.at[pl.ds(0, nchunks)])
            ng = nchunks // K
            fire_gathers(0, 0, gsem0)

            @pl.loop(0, ng // 2)
            def _(t):
                a = t * (2 * K)
                b = a + K
                drain_gathers(a, 0, gsem0)
                fire_scatters(a, 0, ssem0)

                @pl.when(t > 0)
                def _():
                    drain_scatters(a - K, 1, ssem1)

                fire_gathers(b, 1, gsem1)
                drain_gathers(b, 1, gsem1)
                fire_scatters(b, 1, ssem1)
                drain_scatters(a, 0, ssem0)

                @pl.when(t < ng // 2 - 1)
                def _():
                    fire_gathers(b + K, 0, gsem0)

            drain_scatters(nchunks - K, 1, ssem1)

        pltpu.sync_copy(p_hbm.at[pl.ds(sid * RPS, RPS)],
                        tab_sh.at[pl.ds(sid * RPS, RPS)])
        _zero_buf(bufs.at[0], width)
        for c in range(RPT // CHUNK):
            pltpu.sync_copy(bufs.at[0],
                            acc_sh.at[pl.ds(sid * RPT + c * CHUNK, CHUNK)])
        plsc.subcore_barrier()

        if c0 == c1:
            run(cid * NS * c0 + sid * c0, c0)
        else:
            @pl.when(cid == 0)
            def _():
                run(sid * c0, c0)

            if c1 > 0:
                @pl.when(cid == 1)
                def _():
                    run(NS * c0 + sid * c1, c1)

        plsc.subcore_barrier()
        _copy_out_stripe(acc_sh, bufs.at[0], out_hbm, cid, sid, width)

    return agg


def _tc_matmul_body(x_ref, w_ref, h_ref):
    h_ref[...] = jnp.dot(x_ref[...], w_ref[...],
                         preferred_element_type=jnp.float32)


def _tc_scale_body(deg0, deg1, h_ref, p_ref):
    dis = lax.rsqrt(deg0[...] + deg1[...] + 1.0)
    p_ref[...] = dis * h_ref[...]


def _tc_mid_body(deg0, deg1, acc_ref, p_ref, w_ref, b_ref, out_ref):
    dis = lax.rsqrt(deg0[...] + deg1[...] + 1.0)
    s = acc_ref[0, :N, :] + acc_ref[1, :N, :] + p_ref[...]
    out = jax.nn.relu(dis * s + b_ref[...])
    h = jnp.dot(out, w_ref[...], preferred_element_type=jnp.float32)
    out_ref[...] = dis * h


def _tc_final_body(deg0, deg1, acc_ref, q_ref, wmu_ref, bmu_ref, wlv_ref,
                   blv_ref, mu_ref, lv_ref):
    dis = lax.rsqrt(deg0[...] + deg1[...] + 1.0)
    t = dis * (acc_ref[0, :N, :] + acc_ref[1, :N, :] + q_ref[...])
    mu_ref[...] = (
        jnp.dot(t, wmu_ref[...], preferred_element_type=jnp.float32)
        + bmu_ref[...]
    )
    lv_ref[...] = (
        jnp.dot(t, wlv_ref[...], preferred_element_type=jnp.float32)
        + blv_ref[...]
    )


def _pad2(a, rows, cols):
    out = jnp.zeros((rows, cols), jnp.float32)
    return out.at[: a.shape[0], : a.shape[1]].set(a)


def kernel(x, edge_index, W1, b1, W2, b2, W3, b3, Wmu, bmu, Wlv, blv):
    ei = edge_index.astype(jnp.int32)
    src = jnp.concatenate([ei[0], jnp.zeros((EP - E,), jnp.int32)])
    dst = jnp.concatenate([ei[1], jnp.full((EP - E,), N, jnp.int32)])
    src2d = src.reshape(EP // CHUNK, CHUNK)
    dst2d = dst.reshape(EP // CHUNK, CHUNK)

    iota2d = jnp.arange(DEGR, dtype=jnp.int32).reshape(DEGR // CHUNK, CHUNK)
    degs = _make_sc_degree_hist()(dst2d, iota2d)
    degs = degs.reshape(NC, DEGR * LANES)[:, :, None]
    _sc_agg64 = _make_sc_agg(64, 88, 72, 1)
    _sc_agg16 = _make_sc_agg(16, 88, 72, 4)
    deg0 = degs[0, :N]
    deg1 = degs[1, :N]

    h1 = pl.pallas_call(
        _tc_matmul_body,
        out_shape=jax.ShapeDtypeStruct((N, 64), jnp.float32),
    )(x, W1)
    p1 = pl.pallas_call(
        _tc_scale_body,
        out_shape=jax.ShapeDtypeStruct((N, 64), jnp.float32),
    )(deg0, deg1, h1)
    acc1 = _sc_agg64(p1, src2d, dst2d)

    w2p = _pad2(W2, 64, 16)
    p2 = pl.pallas_call(
        _tc_mid_body,
        out_shape=jax.ShapeDtypeStruct((N, 16), jnp.float32),
    )(deg0, deg1, acc1, p1, w2p, b1.reshape(1, 64))
    acc2 = _sc_agg16(p2, src2d, dst2d)

    w3p = _pad2(W3, 16, 16)
    b2p = _pad2(b2.reshape(1, 8), 1, 16)
    p3 = pl.pallas_call(
        _tc_mid_body,
        out_shape=jax.ShapeDtypeStruct((N, 16), jnp.float32),
    )(deg0, deg1, acc2, p2, w3p, b2p)
    acc3 = _sc_agg16(p3, src2d, dst2d)

    eye = jnp.eye(16, dtype=jnp.float32)
    b3p = _pad2(b3.reshape(1, 4), 1, 16)
    q = pl.pallas_call(
        _tc_mid_body,
        out_shape=jax.ShapeDtypeStruct((N, 16), jnp.float32),
    )(deg0, deg1, acc3, p3, eye, b3p)
    acc4 = _sc_agg16(q, src2d, dst2d)

    wmup = _pad2(Wmu, 16, 2)
    wlvp = _pad2(Wlv, 16, 2)
    mu, lv = pl.pallas_call(
        _tc_final_body,
        out_shape=[
            jax.ShapeDtypeStruct((N, 2), jnp.float32),
            jax.ShapeDtypeStruct((N, 2), jnp.float32),
        ],
    )(deg0, deg1, acc4, q, wmup, bmu.reshape(1, 2),
      wlvp, blv.reshape(1, 2))
    return (mu, lv)


# speedup vs baseline: 1.0370x; 1.0001x over previous
import functools

import jax
import jax.numpy as jnp
from jax import lax
from jax.experimental import pallas as pl
from jax.experimental.pallas import tpu as pltpu
from jax.experimental.pallas import tpu_sc as plsc

N = 10000
E = 320000
NC = 2
NS = 16
LANES = 16
NW = NC * NS
CHUNK = 128
CPT = 80
EPT = CHUNK * CPT
EP = EPT * NW
RPT = 640
NACC = RPT * NS

def _zero_buf(buf, width):
    @pl.loop(0, CHUNK)
    def _(r):
        for c in range(width // LANES):
            buf[r, pl.ds(c * LANES, LANES)] = jnp.zeros((LANES,), jnp.float32)


def _copy_out_stripe(acc_sh, buf, out_hbm, cid, sid, width):
    for c in range(RPT // CHUNK):
        row = sid * RPT + c * CHUNK
        pltpu.sync_copy(acc_sh.at[pl.ds(row, CHUNK)], buf)
        pltpu.sync_copy(buf, out_hbm.at[cid, pl.ds(row, CHUNK)])


DEGR = 640


@functools.cache
def _make_sc_degree_hist():
    STR = DEGR // NS

    @functools.partial(
        pl.kernel,
        out_type=jax.ShapeDtypeStruct((NC, DEGR, LANES), jnp.float32),
        mesh=plsc.VectorSubcoreMesh(core_axis_name="c", subcore_axis_name="s"),
        scratch_types=[
            pltpu.VMEM((CPT, CHUNK), jnp.int32),
            pltpu.VMEM((DEGR, LANES), jnp.float32),
            pltpu.VMEM((DEGR // CHUNK, CHUNK), jnp.int32),
            pltpu.VMEM_SHARED((DEGR, LANES), jnp.float32),
        ],
        compiler_params=pltpu.CompilerParams(use_tc_tiling_on_sc=False,
                                             needs_layout_passes=False),
    )
    def degree(dst_hbm, iota_hbm, out_hbm, dst_v, deg_v, iota_v, acc_sh):
        cid = lax.axis_index("c")
        sid = lax.axis_index("s")
        wid = cid * NS + sid
        pltpu.sync_copy(dst_hbm.at[pl.ds(wid * CPT, CPT)], dst_v)
        pltpu.sync_copy(iota_hbm, iota_v)

        @pl.loop(0, DEGR)
        def _(r):
            deg_v[r, :] = jnp.zeros((LANES,), jnp.float32)

        pltpu.sync_copy(deg_v.at[pl.ds(0, STR)],
                        acc_sh.at[pl.ds(sid * STR, STR)])
        plsc.subcore_barrier()

        ones = jnp.full((LANES,), 1.0, jnp.float32)

        @pl.loop(0, CPT)
        def _(j):
            for k in range(CHUNK // LANES):
                idx = dst_v[j, pl.ds(k * LANES, LANES)]
                row = lax.shift_right_logical(idx, 4)
                col = lax.bitwise_and(idx, 15)
                plsc.addupdate_scatter(deg_v, [row, col], ones)

        for c in range(DEGR // CHUNK):
            pltpu.sync_copy(deg_v.at[pl.ds(c * CHUNK, CHUNK)],
                            acc_sh.at[iota_v.at[c]], add=True)
        plsc.subcore_barrier()
        pltpu.sync_copy(acc_sh.at[pl.ds(sid * STR, STR)],
                        deg_v.at[pl.ds(0, STR)])
        pltpu.sync_copy(deg_v.at[pl.ds(0, STR)],
                        out_hbm.at[cid, pl.ds(sid * STR, STR)])

    return degree


@functools.cache
def _make_sc_agg(width, c0=CPT, c1=CPT, K=4):
    cmax = max(c0, c1)
    RPS = N // NS

    @functools.partial(
        pl.kernel,
        out_type=jax.ShapeDtypeStruct((NC, NACC, width), jnp.float32),
        mesh=plsc.VectorSubcoreMesh(core_axis_name="c", subcore_axis_name="s"),
        scratch_types=[
            pltpu.VMEM((cmax, CHUNK), jnp.int32),
            pltpu.VMEM((cmax, CHUNK), jnp.int32),
            pltpu.VMEM((2 * K, CHUNK, width), jnp.float32),
            pltpu.VMEM_SHARED((NACC, width), jnp.float32),
            pltpu.VMEM_SHARED((N, width), jnp.float32),
            pltpu.SemaphoreType.DMA,
            pltpu.SemaphoreType.DMA,
            pltpu.SemaphoreType.DMA,
            pltpu.SemaphoreType.DMA,
        ],
        compiler_params=pltpu.CompilerParams(use_tc_tiling_on_sc=False),
    )
    def agg(p_hbm, src_hbm, dst_hbm, out_hbm, src_v, dst_v, bufs, acc_sh,
            tab_sh, gsem0, gsem1, ssem0, ssem1):
        cid = lax.axis_index("c")
        sid = lax.axis_index("s")

        def fire_gathers(base, bank, sem):
            for k in range(K):
                pltpu.async_copy(tab_sh.at[src_v.at[base + k]],
                                 bufs.at[bank * K + k], sem)

        def drain_gathers(base, bank, sem):
            for k in range(K):
                pltpu.make_async_copy(tab_sh.at[src_v.at[base + k]],
                                      bufs.at[bank * K + k], sem).wait()

        def fire_scatters(base, bank, sem):
            for k in range(K):
                pltpu.async_copy(bufs.at[bank * K + k],
                                 acc_sh.at[dst_v.at[base + k]], sem, add=True)

        def drain_scatters(base, bank, sem):
            for k in range(K):
                pltpu.make_async_copy(bufs.at[bank * K + k],
                                      acc_sh.at[dst_v.at[base + k]],
                                      sem).wait()

        def run(hbm_base, nchunks):
            pltpu.sync_copy(src_hbm.at[pl.ds(hbm_base, nchunks)],
                            src_v.at[pl.ds(0, nchunks)])
            pltpu.sync_copy(dst_hbm.at[pl.ds(hbm_base, nchunks)],
                            dst_v.at[pl.ds(0, nchunks)])
            ng = nchunks // K
            fire_gathers(0, 0, gsem0)

            @pl.loop(0, ng // 2)
            def _(t):
                a = t * (2 * K)
                b = a + K
                drain_gathers(a, 0, gsem0)
                fire_scatters(a, 0, ssem0)

                @pl.when(t > 0)
                def _():
                    drain_scatters(a - K, 1, ssem1)

                fire_gathers(b, 1, gsem1)
                drain_gathers(b, 1, gsem1)
                fire_scatters(b, 1, ssem1)
                drain_scatters(a, 0, ssem0)

                @pl.when(t < ng // 2 - 1)
                def _():
                    fire_gathers(b + K, 0, gsem0)

            drain_scatters(nchunks - K, 1, ssem1)

        pltpu.sync_copy(p_hbm.at[pl.ds(sid * RPS, RPS)],
                        tab_sh.at[pl.ds(sid * RPS, RPS)])
        _zero_buf(bufs.at[0], width)
        for c in range(RPT // CHUNK):
            pltpu.sync_copy(bufs.at[0],
                            acc_sh.at[pl.ds(sid * RPT + c * CHUNK, CHUNK)])
        plsc.subcore_barrier()

        if c0 == c1:
            run(cid * NS * c0 + sid * c0, c0)
        else:
            @pl.when(cid == 0)
            def _():
                run(sid * c0, c0)

            if c1 > 0:
                @pl.when(cid == 1)
                def _():
                    run(NS * c0 + sid * c1, c1)

        plsc.subcore_barrier()
        _copy_out_stripe(acc_sh, bufs.at[0], out_hbm, cid, sid, width)

    return agg


def _tc_matmul_body(x_ref, w_ref, h_ref):
    h_ref[...] = jnp.dot(x_ref[...], w_ref[...],
                         preferred_element_type=jnp.float32)


def _tc_scale_body(deg0, deg1, h_ref, p_ref):
    dis = lax.rsqrt(deg0[...] + deg1[...] + 1.0)
    p_ref[...] = dis * h_ref[...]


def _tc_mid_body(deg0, deg1, acc_ref, p_ref, w_ref, b_ref, out_ref):
    dis = lax.rsqrt(deg0[...] + deg1[...] + 1.0)
    s = acc_ref[0, :N, :] + acc_ref[1, :N, :] + p_ref[...]
    out = jax.nn.relu(dis * s + b_ref[...])
    h = jnp.dot(out, w_ref[...], preferred_element_type=jnp.float32)
    out_ref[...] = dis * h


def _tc_final_body(deg0, deg1, acc_ref, q_ref, wmu_ref, bmu_ref, wlv_ref,
                   blv_ref, mu_ref, lv_ref):
    dis = lax.rsqrt(deg0[...] + deg1[...] + 1.0)
    t = dis * (acc_ref[0, :N, :] + acc_ref[1, :N, :] + q_ref[...])
    mu_ref[...] = (
        jnp.dot(t, wmu_ref[...], preferred_element_type=jnp.float32)
        + bmu_ref[...]
    )
    lv_ref[...] = (
        jnp.dot(t, wlv_ref[...], preferred_element_type=jnp.float32)
        + blv_ref[...]
    )


def _pad2(a, rows, cols):
    out = jnp.zeros((rows, cols), jnp.float32)
    return out.at[: a.shape[0], : a.shape[1]].set(a)


def kernel(x, edge_index, W1, b1, W2, b2, W3, b3, Wmu, bmu, Wlv, blv):
    ei = edge_index.astype(jnp.int32)
    src = jnp.concatenate([ei[0], jnp.zeros((EP - E,), jnp.int32)])
    dst = jnp.concatenate([ei[1], jnp.full((EP - E,), N, jnp.int32)])
    src2d = src.reshape(EP // CHUNK, CHUNK)
    dst2d = dst.reshape(EP // CHUNK, CHUNK)

    iota2d = jnp.arange(DEGR, dtype=jnp.int32).reshape(DEGR // CHUNK, CHUNK)
    degs = _make_sc_degree_hist()(dst2d, iota2d)
    degs = degs.reshape(NC, DEGR * LANES)[:, :, None]
    _sc_agg64 = _make_sc_agg(64, 88, 72, 1)
    _sc_agg16 = _make_sc_agg(16, 88, 72, 4)
    deg0 = degs[0, :N]
    deg1 = degs[1, :N]

    h1 = pl.pallas_call(
        _tc_matmul_body,
        out_shape=jax.ShapeDtypeStruct((N, 64), jnp.float32),
    )(x, W1)
    p1 = pl.pallas_call(
        _tc_scale_body,
        out_shape=jax.ShapeDtypeStruct((N, 64), jnp.float32),
    )(deg0, deg1, h1)
    acc1 = _sc_agg64(p1, src2d, dst2d)

    w2p = _pad2(W2, 64, 16)
    p2 = pl.pallas_call(
        _tc_mid_body,
        out_shape=jax.ShapeDtypeStruct((N, 16), jnp.float32),
    )(deg0, deg1, acc1, p1, w2p, b1.reshape(1, 64))
    acc2 = _sc_agg16(p2, src2d, dst2d)

    w3p = _pad2(W3, 16, 16)
    b2p = _pad2(b2.reshape(1, 8), 1, 16)
    p3 = pl.pallas_call(
        _tc_mid_body,
        out_shape=jax.ShapeDtypeStruct((N, 16), jnp.float32),
    )(deg0, deg1, acc2, p2, w3p, b2p)
    acc3 = _sc_agg16(p3, src2d, dst2d)

    eye = jnp.eye(16, dtype=jnp.float32)
    b3p = _pad2(b3.reshape(1, 4), 1, 16)
    q = pl.pallas_call(
        _tc_mid_body,
        out_shape=jax.ShapeDtypeStruct((N, 16), jnp.float32),
    )(deg0, deg1, acc3, p3, eye, b3p)
    acc4 = _sc_agg16(q, src2d, dst2d)

    wmup = _pad2(Wmu, 16, 2)
    wlvp = _pad2(Wlv, 16, 2)
    mu, lv = pl.pallas_call(
        _tc_final_body,
        out_shape=[
            jax.ShapeDtypeStruct((N, 2), jnp.float32),
            jax.ShapeDtypeStruct((N, 2), jnp.float32),
        ],
    )(deg0, deg1, acc4, q, wmup, bmu.reshape(1, 2),
      wlvp, blv.reshape(1, 2))
    return (mu, lv)
